# Initial kernel scaffold; baseline (speedup 1.0000x reference)
import jax
import jax.numpy as jnp
from jax.experimental import pallas as pl


def kernel(x, edge_index, c, edge_attr, batch, colouring2graph, We, be, eps, W1, b1, g1, bb1, W2, b2, Wp, bp):
    raise NotImplementedError("write your pallas kernel here")


# speedup vs baseline: 4.2107x; 4.2107x over previous
import functools

import jax
import jax.numpy as jnp
from jax import lax
from jax.experimental import pallas as pl
from jax.experimental.pallas import tpu as pltpu
from jax.experimental.pallas import tpu_sc as plsc

_NCORES = 2
_NSUB = 16
_NW = _NCORES * _NSUB
_LANES = 16


def _edge_embed(ea, w, b):
    E, ED = ea.shape
    D = w.shape[1]
    BE = 4000
    grid = E // BE

    def body(ea_ref, w_ref, b_ref, o_ref):
        o_ref[...] = (
            jnp.dot(ea_ref[...], w_ref[...], preferred_element_type=jnp.float32)
            + b_ref[...]
        )

    return pl.pallas_call(
        body,
        grid=(grid,),
        in_specs=[
            pl.BlockSpec((BE, ED), lambda i: (i, 0)),
            pl.BlockSpec((ED, D), lambda i: (0, 0)),
            pl.BlockSpec((1, D), lambda i: (0, 0)),
        ],
        out_specs=pl.BlockSpec((BE, D), lambda i: (i, 0)),
        out_shape=jax.ShapeDtypeStruct((E, D), jnp.float32),
    )(ea, w, b)


def _sc_aggregate(x, e, src1, dst1):
    N, D = x.shape
    E = src1.shape[0]
    EW = E // _NW
    K = 40
    NCH = EW // K
    R = 2
    M = 5
    assert EW % K == 0 and NCH % 2 == 0 and NCH >= 8
    CP = 1000
    NZW = N // CP
    assert N % CP == 0 and NZW <= _NSUB

    mesh = plsc.VectorSubcoreMesh(
        core_axis_name="c", subcore_axis_name="s",
        num_cores=_NCORES, num_subcores=_NSUB,
    )

    @functools.partial(
        pl.kernel,
        out_type=[jax.ShapeDtypeStruct((N, D), jnp.float32)] * 2,
        mesh=mesh,
        scratch_types=[
            pltpu.VMEM((M, K), jnp.int32),
            pltpu.VMEM((M, K), jnp.int32),
            pltpu.VMEM((R * K, D), jnp.float32),
            pltpu.VMEM((R * K, D), jnp.float32),
            pltpu.VMEM((R * K, D), jnp.float32),
            pltpu.VMEM_SHARED((N, D), jnp.float32),
            pltpu.SemaphoreType.DMA,
            pltpu.SemaphoreType.DMA,
            pltpu.SemaphoreType.DMA,
            pltpu.SemaphoreType.DMA,
            pltpu.SemaphoreType.DMA,
            pltpu.SemaphoreType.DMA,
            pltpu.SemaphoreType.DMA,
            pltpu.SemaphoreType.DMA,
        ],
    )
    def k(x_hbm, e_hbm, src_hbm, dst_hbm, out0, out1,
          srcr, dstr, rows, ebuf, msg, aggr,
          gs0, gs1, es0, es1, ss0, ss1, is0, is1):
        cid = lax.axis_index("c")
        sid = lax.axis_index("s")
        wid = cid * _NSUB + sid
        gsem = [gs0, gs1]
        esem = [es0, es1]
        ssem = [ss0, ss1]
        isem = [is0, is1]
        base = wid * EW

        def zrow(r, _):
            for j in range(D // _LANES):
                msg[r, pl.ds(j * _LANES, _LANES)] = jnp.zeros(
                    (_LANES,), jnp.float32)
            return 0
        lax.fori_loop(0, R * K, zrow, 0)

        @pl.when(sid < NZW)
        def _():
            r0 = sid * CP
            for t in range(CP // (R * K)):
                pltpu.sync_copy(msg, aggr.at[pl.ds(r0 + t * R * K, R * K)])
            rem = CP % (R * K)
            if rem:
                pltpu.sync_copy(msg.at[pl.ds(0, rem)],
                                aggr.at[pl.ds(r0 + CP - rem, rem)])
        plsc.subcore_barrier()

        def issue_idx(g, p):
            pltpu.async_copy(src_hbm.at[pl.ds(base + g * K, K)],
                             srcr.at[g % M], isem[p])
            pltpu.async_copy(dst_hbm.at[pl.ds(base + g * K, K)],
                             dstr.at[g % M], isem[p])

        def wait_idx(g, p):
            pltpu.make_async_copy(src_hbm.at[pl.ds(base + g * K, K)],
                                  srcr.at[g % M], isem[p]).wait()
            pltpu.make_async_copy(dst_hbm.at[pl.ds(base + g * K, K)],
                                  dstr.at[g % M], isem[p]).wait()

        def issue_gather(g, b):
            pltpu.async_copy(x_hbm.at[srcr.at[g % M]],
                             rows.at[pl.ds(b * K, K)], gsem[b])

        def wait_gather(g, b):
            pltpu.make_async_copy(x_hbm.at[srcr.at[g % M]],
                                  rows.at[pl.ds(b * K, K)], gsem[b]).wait()

        def issue_e(g, b):
            pltpu.async_copy(e_hbm.at[pl.ds(base + g * K, K)],
                             ebuf.at[pl.ds(b * K, K)], esem[b])

        def wait_e(g, b):
            pltpu.make_async_copy(e_hbm.at[pl.ds(base + g * K, K)],
                                  ebuf.at[pl.ds(b * K, K)], esem[b]).wait()

        def issue_scatter(g, b):
            pltpu.async_copy(msg.at[pl.ds(b * K, K)], aggr.at[dstr.at[g % M]],
                             ssem[b], add=True)

        def wait_scatter(g, b):
            pltpu.make_async_copy(msg.at[pl.ds(b * K, K)],
                                  aggr.at[dstr.at[g % M]], ssem[b]).wait()

        def compute(b):
            def row_body(r, _):
                for j in range(D // _LANES):
                    sl = pl.ds(j * _LANES, _LANES)
                    msg[b * K + r, sl] = jnp.maximum(
                        rows[b * K + r, sl] + ebuf[b * K + r, sl], 0.0)
                return 0
            lax.fori_loop(0, K, row_body, 0)

        def body(g, b, wait_sc, next2, next3):
            if next2:
                wait_idx(g + 2, b)
            wait_gather(g, b)
            wait_e(g, b)
            if wait_sc:
                wait_scatter(g - 2, b)
            compute(b)
            issue_scatter(g, b)
            if next2:
                issue_gather(g + 2, b)
                issue_e(g + 2, b)
            if next3:
                issue_idx(g + 3, 1 - b)

        issue_idx(0, 0)
        issue_idx(1, 1)
        wait_idx(0, 0)
        wait_idx(1, 1)
        for b in range(R):
            issue_gather(b, b)
            issue_e(b, b)
        issue_idx(2, 0)

        body(0, 0, wait_sc=False, next2=True, next3=True)
        body(1, 1, wait_sc=False, next2=True, next3=True)

        def outer(kk, _):
            g0 = kk * 2
            for b in range(R):
                body(g0 + b, b, wait_sc=True, next2=True, next3=True)
            return 0
        lax.fori_loop(1, (NCH - 4) // 2, outer, 0)

        body(NCH - 4, 0, wait_sc=True, next2=True, next3=True)
        body(NCH - 3, 1, wait_sc=True, next2=True, next3=False)
        body(NCH - 2, 0, wait_sc=True, next2=False, next3=False)
        body(NCH - 1, 1, wait_sc=True, next2=False, next3=False)
        wait_scatter(NCH - 2, 0)
        wait_scatter(NCH - 1, 1)

        plsc.subcore_barrier()

        @pl.when(jnp.logical_and(sid < NZW, cid == 0))
        def _():
            pltpu.sync_copy(aggr.at[pl.ds(sid * CP, CP)],
                            out0.at[pl.ds(sid * CP, CP)])

        @pl.when(jnp.logical_and(sid < NZW, cid == 1))
        def _():
            pltpu.sync_copy(aggr.at[pl.ds(sid * CP, CP)],
                            out1.at[pl.ds(sid * CP, CP)])

    return k(x, e, src1, dst1)


def _mlp_in(cur, a0, a1, cc, epsb, w1a, w1b, b1):
    N, D = cur.shape
    CD = cc.shape[1]
    BN_ = 400
    grid = N // BN_

    def body(cur_ref, a0_ref, a1_ref, cc_ref, epsb_ref, w1a_ref, w1b_ref,
             b1_ref, h1_ref, st_ref, acc_ref):
        i = pl.program_id(0)
        z = cur_ref[...] * epsb_ref[...] + a0_ref[...] + a1_ref[...]
        h = (jnp.dot(z, w1a_ref[...], preferred_element_type=jnp.float32)
             + jnp.dot(cc_ref[...], w1b_ref[...], preferred_element_type=jnp.float32)
             + b1_ref[...])
        h1_ref[...] = h
        blk = jnp.concatenate(
            [jnp.sum(h, axis=0, keepdims=True),
             jnp.sum(h * h, axis=0, keepdims=True)], axis=0)

        @pl.when(i == 0)
        def _():
            acc_ref[...] = blk

        @pl.when(i > 0)
        def _():
            acc_ref[...] += blk

        @pl.when(i == grid - 1)
        def _():
            st_ref[...] = acc_ref[...]

    return pl.pallas_call(
        body,
        grid=(grid,),
        in_specs=[
            pl.BlockSpec((BN_, D), lambda i: (i, 0)),
            pl.BlockSpec((BN_, D), lambda i: (i, 0)),
            pl.BlockSpec((BN_, D), lambda i: (i, 0)),
            pl.BlockSpec((BN_, CD), lambda i: (i, 0)),
            pl.BlockSpec((1, D), lambda i: (0, 0)),
            pl.BlockSpec((D, D), lambda i: (0, 0)),
            pl.BlockSpec((CD, D), lambda i: (0, 0)),
            pl.BlockSpec((1, D), lambda i: (0, 0)),
        ],
        out_specs=[
            pl.BlockSpec((BN_, D), lambda i: (i, 0)),
            pl.BlockSpec((2, D), lambda i: (0, 0)),
        ],
        out_shape=[
            jax.ShapeDtypeStruct((N, D), jnp.float32),
            jax.ShapeDtypeStruct((2, D), jnp.float32),
        ],
        scratch_shapes=[pltpu.VMEM((2, D), jnp.float32)],
    )(cur, a0, a1, cc, epsb, w1a, w1b, b1)


def _mlp_out(h1, stats, g1, bb1, w2, b2, relu_out):
    N, D = h1.shape
    BN_ = 400
    grid = N // BN_
    inv_n = 1.0 / N

    def body(h1_ref, st_ref, g1_ref, bb1_ref, w2_ref, b2_ref, o_ref):
        st = st_ref[...]
        mean = st[0:1] * inv_n
        var = st[1:2] * inv_n - mean * mean
        scale = lax.rsqrt(var + 1e-5) * g1_ref[...]
        hn = (h1_ref[...] - mean) * scale + bb1_ref[...]
        hr = jnp.maximum(hn, 0.0)
        o = jnp.dot(hr, w2_ref[...], preferred_element_type=jnp.float32) + b2_ref[...]
        if relu_out:
            o = jnp.maximum(o, 0.0)
        o_ref[...] = o

    return pl.pallas_call(
        body,
        grid=(grid,),
        in_specs=[
            pl.BlockSpec((BN_, D), lambda i: (i, 0)),
            pl.BlockSpec((2, D), lambda i: (0, 0)),
            pl.BlockSpec((1, D), lambda i: (0, 0)),
            pl.BlockSpec((1, D), lambda i: (0, 0)),
            pl.BlockSpec((D, D), lambda i: (0, 0)),
            pl.BlockSpec((1, D), lambda i: (0, 0)),
        ],
        out_specs=pl.BlockSpec((BN_, D), lambda i: (i, 0)),
        out_shape=jax.ShapeDtypeStruct((N, D), jnp.float32),
    )(h1, stats, g1, bb1, w2, b2)


def _readout(cur, batch3, col3, wp, bp, nc, ng):
    N, D = cur.shape
    OUT = wp.shape[1]
    BN_ = 400
    grid = N // BN_

    def body(cur_ref, b_ref, col_ref, wp_ref, bp_ref, y_ref, acc_ref):
        i = pl.program_id(0)
        b = b_ref[0]
        oh = (lax.broadcasted_iota(jnp.int32, (nc, BN_), 0) == b).astype(jnp.float32)
        blk = jnp.dot(oh, cur_ref[...], preferred_element_type=jnp.float32)

        @pl.when(i == 0)
        def _():
            acc_ref[...] = blk

        @pl.when(i > 0)
        def _():
            acc_ref[...] += blk

        @pl.when(i == grid - 1)
        def _():
            ys = jnp.dot(acc_ref[...], wp_ref[...],
                         preferred_element_type=jnp.float32) + bp_ref[...]
            g = col_ref[0]
            ohg = (lax.broadcasted_iota(jnp.int32, (ng, nc), 0) == g).astype(jnp.float32)
            sums = jnp.dot(ohg, ys, preferred_element_type=jnp.float32)
            counts = jnp.sum(ohg, axis=1, keepdims=True)
            y_ref[...] = sums / jnp.maximum(counts, 1.0)

    return pl.pallas_call(
        body,
        grid=(grid,),
        in_specs=[
            pl.BlockSpec((BN_, D), lambda i: (i, 0)),
            pl.BlockSpec((1, 1, BN_), lambda i: (i, 0, 0)),
            pl.BlockSpec((1, 1, nc), lambda i: (0, 0, 0)),
            pl.BlockSpec((D, OUT), lambda i: (0, 0)),
            pl.BlockSpec((1, OUT), lambda i: (0, 0)),
        ],
        out_specs=pl.BlockSpec((ng, OUT), lambda i: (0, 0)),
        out_shape=jax.ShapeDtypeStruct((ng, OUT), jnp.float32),
        scratch_shapes=[pltpu.VMEM((nc, D), jnp.float32)],
    )(cur, batch3, col3, wp, bp)


def kernel(x, edge_index, c, edge_attr, batch, colouring2graph,
           We, be, eps, W1, b1, g1, bb1, W2, b2, Wp, bp):
    N, D = x.shape
    E = edge_index.shape[1]
    L = We.shape[0]
    NC = int(colouring2graph.shape[0])
    NG = 4

    src1 = edge_index[0]
    dst1 = edge_index[1]

    BN_ = 400
    batch3 = batch.reshape(N // BN_, 1, BN_)
    col3 = colouring2graph.reshape(1, 1, NC)

    cur = x
    for l in range(L):
        e = _edge_embed(edge_attr, We[l], be[l].reshape(1, D))
        a0, a1 = _sc_aggregate(cur, e, src1, dst1)
        epsb = (1.0 + eps[l]) * jnp.ones((1, D), jnp.float32)
        h1, stats = _mlp_in(cur, a0, a1, c, epsb,
                            W1[l, :D], W1[l, D:], b1[l].reshape(1, D))
        cur = _mlp_out(h1, stats, g1[l].reshape(1, D), bb1[l].reshape(1, D),
                       W2[l], b2[l].reshape(1, D), relu_out=(l != L - 1))

    return _readout(cur, batch3, col3, Wp, bp.reshape(1, -1), NC, NG)
